# baseline (speedup 1.0000x reference)
import jax
import jax.numpy as jnp
from jax import lax
from jax.experimental import pallas as pl
from jax.experimental.pallas import tpu as pltpu
from jax.experimental.pallas import tpu_sc as plsc

_B, _S, _H = 4, 8192, 1024
_NW = 32
_PPW = _S // _NW
_C = 8
_CW = _C * _H
_NCHUNK = _PPW // _C


def _sc_add(x2, pe):
    mesh = plsc.VectorSubcoreMesh(core_axis_name="c", subcore_axis_name="s")

    scratch = (
        [pltpu.VMEM((_C, _H), jnp.float32) for _ in range(12)]
        + [pltpu.VMEM((_C, _H), jnp.float32) for _ in range(2)]
        + [pltpu.SemaphoreType.DMA for _ in range(8)]
    )

    @pl.kernel(
        out_type=jax.ShapeDtypeStruct((_B * _S, _H), jnp.float32),
        mesh=mesh,
        scratch_types=scratch,
    )
    def k(x_hbm, pe_hbm, out_hbm, *scr):
        xb = [[scr[q * 4 + b] for b in range(4)] for q in range(3)]
        peb = [scr[12], scr[13]]
        sx = [scr[14], scr[15], scr[16]]
        so = [scr[17], scr[18], scr[19]]
        spe = [scr[20], scr[21]]

        wid = lax.axis_index("s") * 2 + lax.axis_index("c")
        pos_base = wid * _PPW

        def pe_src(ci):
            return pe_hbm.at[pl.ds(pl.multiple_of(pos_base + ci * _C, _C), _C)]

        def x_row(ci, b):
            return pl.multiple_of(b * _S + pos_base + ci * _C, _C)

        def x_src(ci, b):
            return x_hbm.at[pl.ds(x_row(ci, b), _C)]

        def out_dst(ci, b):
            return out_hbm.at[pl.ds(x_row(ci, b), _C)]

        def issue_loads(ci):
            q = ci % 3
            for b in range(_B):
                pltpu.async_copy(x_src(ci, b), xb[q][b], sx[q])

        pltpu.async_copy(pe_src(0), peb[0], spe[0])
        issue_loads(0)
        if _NCHUNK > 1:
            pltpu.async_copy(pe_src(1), peb[1], spe[1])

        for ci in range(_NCHUNK):
            q = ci % 3
            p = ci % 2
            if ci >= 2:
                q2 = (ci - 2) % 3
                for b in range(_B):
                    pltpu.make_async_copy(
                        xb[q2][b], out_dst(ci - 2, b), so[q2]).wait()
            if ci + 1 < _NCHUNK:
                issue_loads(ci + 1)
            for b in range(_B):
                pltpu.make_async_copy(x_src(ci, b), xb[q][b], sx[q]).wait()
            pltpu.make_async_copy(pe_src(ci), peb[p], spe[p]).wait()

            pebp = peb[p]

            for b0 in (0, 2):
                xa, xc = xb[q][b0], xb[q][b0 + 1]

                @plsc.parallel_loop(0, _CW, 16, unroll=8)
                def add_body(i):
                    r = lax.shift_right_logical(i, 10)
                    c = pl.multiple_of(lax.bitwise_and(i, _H - 1), 16)
                    pv = pebp[r, pl.ds(c, 16)]
                    xa[r, pl.ds(c, 16)] = xa[r, pl.ds(c, 16)] + pv
                    xc[r, pl.ds(c, 16)] = xc[r, pl.ds(c, 16)] + pv

                pltpu.async_copy(xb[q][b0], out_dst(ci, b0), so[q])
                pltpu.async_copy(xb[q][b0 + 1], out_dst(ci, b0 + 1), so[q])

            if ci + 2 < _NCHUNK:
                pltpu.async_copy(pe_src(ci + 2), peb[p], spe[p])

        for ci in (_NCHUNK - 2, _NCHUNK - 1):
            for b in range(_B):
                pltpu.make_async_copy(
                    xb[ci % 3][b], out_dst(ci, b), so[ci % 3]).wait()

    return k(x2, pe)


def kernel(x, pe):
    B, S, H = x.shape
    out = _sc_add(x.reshape(B * S, H), pe)
    return out.reshape(B, S, H)
